# Initial kernel scaffold; baseline (speedup 1.0000x reference)
import jax
import jax.numpy as jnp
from jax.experimental import pallas as pl


def kernel(x, edge_index, edge_label_index, W1, b1, W2, b2):
    raise NotImplementedError("write your pallas kernel here")


# speedup vs baseline: 2.1351x; 2.1351x over previous
import functools

import jax
import jax.numpy as jnp
from jax import lax
from jax.experimental import pallas as pl
from jax.experimental.pallas import tpu as pltpu
from jax.experimental.pallas import tpu_sc as plsc

NT = 32
NLANE = 16
RPT = 320
NPAD = NT * RPT
CAP = 12288
RC = 128
DC = 128

_SC_PARAMS = pltpu.CompilerParams(needs_layout_passes=False)


def _mesh():
    return plsc.VectorSubcoreMesh(core_axis_name="c", subcore_axis_name="s")


def _wid():
    return lax.axis_index("s") * 2 + lax.axis_index("c")


def _route_kernel(E, CE, src_hbm, dst_hbm, osrc, odst, ocnt, odeg,
                  ebs, ebd, lsrc, ldst, degv, cntv):
    wid = _wid()
    lo = wid * RPT
    iota = lax.iota(jnp.int32, NLANE)
    ones_f = jnp.ones((NLANE,), jnp.float32)
    zeros_i = jnp.zeros((NLANE,), jnp.int32)
    pad_d = jnp.full((NLANE,), RPT, jnp.int32)

    for i in range((RPT + NLANE) // NLANE):
        degv[pl.ds(i * NLANE, NLANE)] = jnp.zeros((NLANE,), jnp.float32)

    def group(g, off):
        d16 = ebd[pl.ds(g * NLANE, NLANE)]
        s16 = ebs[pl.ds(g * NLANE, NLANE)]
        m = (d16 >= lo) & (d16 < lo + RPT)
        dloc = jnp.where(m, d16 - lo, RPT)
        plsc.addupdate_scatter(degv, [dloc], ones_f)
        pc = plsc.all_reduce_population_count(m)
        ci = plsc.cumsum(m.astype(jnp.int32))
        idx = off + ci - 1
        idx = jnp.where(m, idx, CAP + iota)
        idx = jnp.where(idx >= CAP, CAP + iota, idx)
        plsc.store_scatter(lsrc, [idx], s16)
        plsc.store_scatter(ldst, [idx], dloc)
        return off + pc

    def chunk(c, off):
        pltpu.sync_copy(src_hbm.at[pl.ds(c * CE, CE)], ebs)
        pltpu.sync_copy(dst_hbm.at[pl.ds(c * CE, CE)], ebd)
        return lax.fori_loop(0, CE // NLANE, group, off)

    off = lax.fori_loop(0, E // CE, chunk, jnp.zeros((NLANE,), jnp.int32))
    cnt = jnp.minimum(off, CAP)
    cnt_pad = jnp.bitwise_and(cnt + (RC - 1), -RC)
    for g in range(RC // NLANE):
        idxp = cnt + g * NLANE + iota
        mp = idxp < cnt_pad
        idxp = jnp.where(mp, idxp, CAP + iota)
        plsc.store_scatter(lsrc, [idxp], zeros_i)
        plsc.store_scatter(ldst, [idxp], pad_d)
    cntv[...] = cnt_pad
    pltpu.sync_copy(lsrc.at[pl.ds(0, CAP)], osrc.at[pl.ds(wid * CAP, CAP)])
    pltpu.sync_copy(ldst.at[pl.ds(0, CAP)], odst.at[pl.ds(wid * CAP, CAP)])
    pltpu.sync_copy(cntv, ocnt.at[pl.ds(wid * NLANE, NLANE)])
    pltpu.sync_copy(degv.at[pl.ds(0, RPT)], odeg.at[pl.ds(wid * RPT, RPT)])


def _route(src, dst):
    E = src.shape[0]
    CE = 16000 if E % 16000 == 0 else E
    kern = functools.partial(
        pl.kernel,
        out_type=(
            jax.ShapeDtypeStruct((NT * CAP,), jnp.int32),
            jax.ShapeDtypeStruct((NT * CAP,), jnp.int32),
            jax.ShapeDtypeStruct((NT * NLANE,), jnp.int32),
            jax.ShapeDtypeStruct((NT * RPT,), jnp.float32),
        ),
        mesh=_mesh(),
        compiler_params=_SC_PARAMS,
        scratch_types=[
            pltpu.VMEM((CE,), jnp.int32),
            pltpu.VMEM((CE,), jnp.int32),
            pltpu.VMEM((CAP + NLANE,), jnp.int32),
            pltpu.VMEM((CAP + NLANE,), jnp.int32),
            pltpu.VMEM((RPT + NLANE,), jnp.float32),
            pltpu.VMEM((NLANE,), jnp.int32),
        ],
    )(functools.partial(_route_kernel, E, CE))
    return kern(src, dst)


def _agg_kernel(H, hs_hbm, osrc, odst, ocnt, out_hbm,
                lsrc, ldst, cntv, acc, rows, sem):
    wid = _wid()
    lo = wid * RPT
    iota = lax.iota(jnp.int32, NLANE)
    pltpu.sync_copy(osrc.at[pl.ds(wid * CAP, CAP)], lsrc)
    pltpu.sync_copy(odst.at[pl.ds(wid * CAP, CAP)], ldst)
    pltpu.sync_copy(ocnt.at[pl.ds(wid * NLANE, NLANE)], cntv)
    cnt_pad = jnp.max(cntv[...])

    def zrow(r, _):
        for k in range(H // NLANE):
            acc[r, pl.ds(k * NLANE, NLANE)] = jnp.zeros((NLANE,), jnp.float32)
        return 0

    lax.fori_loop(0, RPT + 1, zrow, 0)

    def chunk(c, _):
        pltpu.async_copy(hs_hbm.at[lsrc.at[pl.ds(c * RC, RC)]], rows, sem).wait()

        def group(g, _):
            d16 = ldst[pl.ds(c * RC + g * NLANE, NLANE)]
            r16 = g * NLANE + iota
            for w in range(H):
                col = jnp.full((NLANE,), w, jnp.int32)
                v = plsc.load_gather(rows, [r16, col])
                plsc.addupdate_scatter(acc, [d16, col], v)
            return 0

        lax.fori_loop(0, RC // NLANE, group, 0)
        return 0

    lax.fori_loop(0, cnt_pad // RC, chunk, 0)
    pltpu.sync_copy(acc.at[pl.ds(0, RPT)], out_hbm.at[pl.ds(lo, RPT)])


def _agg(hs, osrc, odst, ocnt):
    H = hs.shape[1]
    kern = functools.partial(
        pl.kernel,
        out_type=jax.ShapeDtypeStruct((NPAD, H), jnp.float32),
        mesh=_mesh(),
        compiler_params=_SC_PARAMS,
        scratch_types=[
            pltpu.VMEM((CAP,), jnp.int32),
            pltpu.VMEM((CAP,), jnp.int32),
            pltpu.VMEM((NLANE,), jnp.int32),
            pltpu.VMEM((RPT + 1, H), jnp.float32),
            pltpu.VMEM((RC, H), jnp.float32),
            pltpu.SemaphoreType.DMA,
        ],
    )(functools.partial(_agg_kernel, H))
    return kern(hs, osrc, odst, ocnt)


def _decode_kernel(TPL, H, z_hbm, srcl_hbm, dstl_hbm, out_hbm,
                   sidx, didx, arows, brows, tbuf, obuf, sem1, sem2):
    wid = _wid()
    base = wid * TPL
    iota = lax.iota(jnp.int32, NLANE)
    pltpu.sync_copy(srcl_hbm.at[pl.ds(base, TPL)], sidx)
    pltpu.sync_copy(dstl_hbm.at[pl.ds(base, TPL)], didx)
    iota16 = iota * NLANE

    def chunk(c, _):
        ca = pltpu.async_copy(z_hbm.at[sidx.at[pl.ds(c * DC, DC)]], arows, sem1)
        cb = pltpu.async_copy(z_hbm.at[didx.at[pl.ds(c * DC, DC)]], brows, sem2)
        ca.wait()
        cb.wait()

        def group(g, _):
            for j in range(NLANE):
                r = g * NLANE + j
                s = arows[r, pl.ds(0, NLANE)] * brows[r, pl.ds(0, NLANE)]
                for k in range(1, H // NLANE):
                    s = s + (arows[r, pl.ds(k * NLANE, NLANE)]
                             * brows[r, pl.ds(k * NLANE, NLANE)])
                tbuf[pl.ds(j * NLANE, NLANE)] = s
            accv = plsc.load_gather(tbuf, [iota16])
            for i in range(1, NLANE):
                accv = accv + plsc.load_gather(tbuf, [iota16 + i])
            obuf[pl.ds(c * DC + g * NLANE, NLANE)] = accv
            return 0

        lax.fori_loop(0, DC // NLANE, group, 0)
        return 0

    lax.fori_loop(0, TPL // DC, chunk, 0)
    pltpu.sync_copy(obuf, out_hbm.at[pl.ds(base, TPL)])


def _decode(z, srcl, dstl):
    H = z.shape[1]
    LPAD = srcl.shape[0]
    TPL = LPAD // NT
    kern = functools.partial(
        pl.kernel,
        out_type=jax.ShapeDtypeStruct((LPAD,), jnp.float32),
        mesh=_mesh(),
        compiler_params=_SC_PARAMS,
        scratch_types=[
            pltpu.VMEM((TPL,), jnp.int32),
            pltpu.VMEM((TPL,), jnp.int32),
            pltpu.VMEM((DC, H), jnp.float32),
            pltpu.VMEM((DC, H), jnp.float32),
            pltpu.VMEM((NLANE * NLANE,), jnp.float32),
            pltpu.VMEM((TPL,), jnp.float32),
            pltpu.SemaphoreType.DMA,
            pltpu.SemaphoreType.DMA,
        ],
    )(functools.partial(_decode_kernel, TPL, H))
    return kern(z, srcl, dstl)


_BR = 400


def _tc1_body(x_ref, w_ref, cnt_ref, dinv_ref, hs_ref):
    dv = lax.rsqrt(cnt_ref[...] + 1.0)
    h = jnp.dot(x_ref[...], w_ref[...], preferred_element_type=jnp.float32)
    dinv_ref[...] = dv
    hs_ref[...] = h * dv


def _tc1(x, W1, cnt):
    N, D = x.shape
    H = W1.shape[1]
    grid = N // _BR
    return pl.pallas_call(
        _tc1_body,
        grid=(grid,),
        in_specs=[
            pl.BlockSpec((_BR, D), lambda i: (i, 0)),
            pl.BlockSpec((D, H), lambda i: (0, 0)),
            pl.BlockSpec((_BR, 1), lambda i: (i, 0)),
        ],
        out_specs=[
            pl.BlockSpec((_BR, 1), lambda i: (i, 0)),
            pl.BlockSpec((_BR, H), lambda i: (i, 0)),
        ],
        out_shape=[
            jax.ShapeDtypeStruct((N, 1), jnp.float32),
            jax.ShapeDtypeStruct((N, H), jnp.float32),
        ],
    )(x, W1, cnt)


def _tc2_body(acc_ref, hs_ref, dinv_ref, w_ref, b_ref, z_ref, hs2_ref):
    dv = dinv_ref[...]
    z = jnp.maximum(dv * (acc_ref[...] + hs_ref[...]) + b_ref[...], 0.0)
    z_ref[...] = z
    hs2_ref[...] = jnp.dot(z, w_ref[...], preferred_element_type=jnp.float32) * dv


def _tc2(acc1, hs1, dinv, W2, b1):
    N, H = hs1.shape
    grid = N // _BR
    return pl.pallas_call(
        _tc2_body,
        grid=(grid,),
        in_specs=[
            pl.BlockSpec((_BR, H), lambda i: (i, 0)),
            pl.BlockSpec((_BR, H), lambda i: (i, 0)),
            pl.BlockSpec((_BR, 1), lambda i: (i, 0)),
            pl.BlockSpec((H, H), lambda i: (0, 0)),
            pl.BlockSpec((1, H), lambda i: (0, 0)),
        ],
        out_specs=[
            pl.BlockSpec((_BR, H), lambda i: (i, 0)),
            pl.BlockSpec((_BR, H), lambda i: (i, 0)),
        ],
        out_shape=[
            jax.ShapeDtypeStruct((N, H), jnp.float32),
            jax.ShapeDtypeStruct((N, H), jnp.float32),
        ],
    )(acc1, hs1, dinv, W2, b1)


def _tc3_body(acc_ref, hs_ref, dinv_ref, b_ref, z_ref):
    z_ref[...] = dinv_ref[...] * (acc_ref[...] + hs_ref[...]) + b_ref[...]


def _tc3(acc2, hs2, dinv, b2):
    N, H = hs2.shape
    grid = N // _BR
    return pl.pallas_call(
        _tc3_body,
        grid=(grid,),
        in_specs=[
            pl.BlockSpec((_BR, H), lambda i: (i, 0)),
            pl.BlockSpec((_BR, H), lambda i: (i, 0)),
            pl.BlockSpec((_BR, 1), lambda i: (i, 0)),
            pl.BlockSpec((1, H), lambda i: (0, 0)),
        ],
        out_specs=pl.BlockSpec((_BR, H), lambda i: (i, 0)),
        out_shape=jax.ShapeDtypeStruct((N, H), jnp.float32),
    )(acc2, hs2, dinv, b2)


def kernel(x, edge_index, edge_label_index, W1, b1, W2, b2):
    N, D = x.shape
    H = W1.shape[1]
    L = edge_label_index.shape[1]

    src = edge_index[0]
    dst = edge_index[1]
    osrc, odst, ocnt, odeg = _route(src, dst)
    cnt = odeg[:N].reshape(N, 1)

    dinv, hs1 = _tc1(x, W1, cnt)
    acc1 = _agg(hs1, osrc, odst, ocnt)[:N]
    z1, hs2 = _tc2(acc1, hs1, dinv, W2, b1.reshape(1, H))
    acc2 = _agg(hs2, osrc, odst, ocnt)[:N]
    z = _tc3(acc2, hs2, dinv, b2.reshape(1, H))

    blk = NT * DC
    LPAD = ((L + blk - 1) // blk) * blk
    srcl = jnp.concatenate([edge_label_index[0],
                            jnp.zeros((LPAD - L,), jnp.int32)])
    dstl = jnp.concatenate([edge_label_index[1],
                            jnp.zeros((LPAD - L,), jnp.int32)])
    logits = _decode(z, srcl, dstl)
    return logits[:L]
